# baseline (speedup 1.0000x reference)
import jax
import jax.numpy as jnp
from jax import lax
from jax.experimental import pallas as pl

B, L, N = 2, 256, 16
HIDDEN = 128
NH = 4
D_IN = HIDDEN * 2
I_N = 4
HS = HIDDEN // NH
T_TOK = L + N
K = L // N - 1
NP = K + 1
NEG = -1000000.0
BIG_NEG = -3.0e38
GCHUNK = 16


def _softmax_rows(x):
    m = jnp.max(x, axis=1, keepdims=True)
    e = jnp.exp(x - m)
    return e / jnp.sum(e, axis=1, keepdims=True)


def _kth_threshold(pt):
    def step(_, carry):
        m, cum, t = carry
        cur = jnp.max(m, axis=0, keepdims=True)
        t = jnp.where(cum < (NP - 0.5), cur, t)
        eq = m == cur
        cum = cum + jnp.sum(eq.astype(jnp.float32), axis=0, keepdims=True)
        m = jnp.where(eq, BIG_NEG, m)
        return m, cum, t

    cols = pt.shape[1]
    init = (pt, jnp.zeros((1, cols), jnp.float32),
            jnp.full((1, cols), BIG_NEG, jnp.float32))
    _, _, t = lax.fori_loop(0, NP, step, init)
    return t


def _body(x_ref, lab_ref, lt_ref, wq_ref, bq_ref, wk_ref, bk_ref, wv_ref,
          bv_ref, wo_ref, bo_ref, out_ref):
    f32 = jnp.float32
    Wq, Wk, Wv, Wo = wq_ref[...], wk_ref[...], wv_ref[...], wo_ref[...]
    bq, bk, bv, bo = bq_ref[...], bk_ref[...], bv_ref[...], bo_ref[...]

    row_i = lax.broadcasted_iota(jnp.int32, (L, L), 0)
    col_i = lax.broadcasted_iota(jnp.int32, (L, L), 1)
    eye = (row_i == col_i).astype(f32)

    sds, rds, graphs, labels, lts = [], [], [], [], []
    for b in range(B):
        x = x_ref[b]
        q = jnp.dot(x, Wq, preferred_element_type=f32) + bq
        k = jnp.dot(x, Wk, preferred_element_type=f32) + bk
        v = jnp.dot(x, Wv, preferred_element_type=f32) + bv
        ctxs = []
        for h in range(NH):
            sl = slice(h * HS, (h + 1) * HS)
            s = lax.dot_general(q[:, sl], k[:, sl],
                                (((1,), (1,)), ((), ())),
                                preferred_element_type=f32)
            p = _softmax_rows(s)
            ctxs.append(jnp.dot(p, v[:, sl], preferred_element_type=f32))
        ctx = jnp.concatenate(ctxs, axis=1)
        g = jnp.dot(ctx, Wo, preferred_element_type=f32) + bo

        sd = jnp.concatenate([g[:L], x[:L]], axis=1)
        rd = jnp.concatenate([g[L:], x[L:]], axis=1)

        gcols = []
        for j in range(0, L, GCHUNK):
            d = sd[:, None, :] - sd[None, j:j + GCHUNK, :]
            gcols.append(-jnp.sqrt(jnp.sum(d * d, axis=-1) + 1e-06))
        graph = jnp.concatenate(gcols, axis=1) + eye * NEG

        sds.append(sd)
        rds.append(rd)
        graphs.append(graph)
        labels.append(lab_ref[b])
        lts.append(lt_ref[b])

    psums = [jnp.zeros((L, N), f32) for _ in range(B)]
    c1 = (K + 1.0) / (K + 2.0)
    c2 = 1.0 / (K + 2.0)

    for it in range(I_N):
        preds = []
        for b in range(B):
            lab, lt = labels[b], lts[b]
            mask = lax.dot_general(lab, lab, (((1,), (1,)), ((), ())),
                                   preferred_element_type=f32)
            dis = graphs[b] * mask + (1.0 - mask) * NEG
            sample_w = jnp.max(dis, axis=0, keepdims=True)
            ltmp = lt * sample_w + (1.0 - lt) * NEG
            probs = _softmax_rows(ltmp)
            center = c1 * jnp.dot(probs, sds[b],
                                  preferred_element_type=f32) + c2 * rds[b]
            d = sds[b][:, None, :] - center[None, :, :]
            pred = -jnp.sum(d * d, axis=-1)
            preds.append(pred)
            psums[b] = psums[b] + pred

        if it == I_N - 1:
            break

        def onehot_step(_, carry):
            p0, p1, keep0, keep1, done = carry
            thr = _kth_threshold(jnp.concatenate([p0, p1], axis=1))
            m2_0 = (p0 >= thr[:, :N]).astype(f32)
            m2_1 = (p1 >= thr[:, N:]).astype(f32)
            keep0 = jnp.where(done, keep0, m2_0)
            keep1 = jnp.where(done, keep1, m2_1)
            ok0 = jnp.all(jnp.sum(m2_0, axis=1, keepdims=True) < 2.0)
            ok1 = jnp.all(jnp.sum(m2_1, axis=1, keepdims=True) < 2.0)
            done = jnp.logical_or(done, jnp.logical_and(ok0, ok1))

            new_p = []
            for p, m2 in ((p0, m2_0), (p1, m2_1)):
                p1m = p - (1.0 - m2) * (-NEG)
                value = jnp.max(p1m, axis=1, keepdims=True)
                m3 = (p1m >= value).astype(f32)
                m4 = m3 * m2
                s4 = jnp.sum(m4, axis=1, keepdims=True)
                m5 = (s4 > 0.5).astype(f32)
                p2 = p + m5 * NEG
                m6 = (s4 > 1.5).astype(f32)
                m4 = m4 * (1.0 - m6)
                new_p.append(jnp.where(done, p, m4 * p + (1.0 - m4) * p2))
            return new_p[0], new_p[1], keep0, keep1, done

        init = (preds[0], preds[1],
                jnp.zeros((L, N), f32), jnp.zeros((L, N), f32),
                jnp.zeros((), jnp.bool_))
        _, _, keep0, keep1, _ = lax.fori_loop(0, 11, onehot_step, init)
        labels = [keep0, keep1]
        lts = [jnp.transpose(keep0), jnp.transpose(keep1)]

    for b in range(B):
        out_ref[b] = psums[b] * (1.0 / I_N)


@jax.jit
def kernel(samples, relation, label, Wq, bq, Wk, bk, Wv, bv, Wo, bo):
    x = jnp.concatenate([samples, relation], axis=1)
    lt = jnp.swapaxes(label, 1, 2)
    return pl.pallas_call(
        _body,
        out_shape=jax.ShapeDtypeStruct((B, L, N), jnp.float32),
    )(x, label, lt, Wq, bq.reshape(1, HIDDEN), Wk, bk.reshape(1, HIDDEN),
      Wv, bv.reshape(1, HIDDEN), Wo, bo.reshape(1, HIDDEN))


# speedup vs baseline: 2.6002x; 2.6002x over previous
import jax
import jax.numpy as jnp
from jax import lax
from jax.experimental import pallas as pl

B, L, N = 2, 256, 16
HIDDEN = 128
NH = 4
D_IN = HIDDEN * 2
I_N = 4
HS = HIDDEN // NH
T_TOK = L + N
K = L // N - 1
NP = K + 1
NEG = -1000000.0
BIG_NEG = -3.0e38
GCHUNK = 16


def _softmax_rows(x):
    m = jnp.max(x, axis=1, keepdims=True)
    e = jnp.exp(x - m)
    return e / jnp.sum(e, axis=1, keepdims=True)


def _kth_threshold(pt):
    def step(_, carry):
        m, cum, t = carry
        cur = jnp.max(m, axis=1, keepdims=True)
        t = jnp.where(cum < (NP - 0.5), cur, t)
        eq = m == cur
        cum = cum + jnp.sum(eq.astype(jnp.float32), axis=1, keepdims=True)
        m = jnp.where(eq, BIG_NEG, m)
        return m, cum, t

    rows = pt.shape[0]
    init = (pt, jnp.zeros((rows, 1), jnp.float32),
            jnp.full((rows, 1), BIG_NEG, jnp.float32))
    _, _, t = lax.fori_loop(0, NP, step, init)
    return t


def _to_one_hot_stacked(pt_stack):
    f32 = jnp.float32

    def cond(carry):
        j, _, _, done = carry
        return jnp.logical_and(j < 11, jnp.logical_not(done))

    def body(carry):
        j, pt, _, done = carry
        thr = _kth_threshold(pt)
        m2 = (pt >= thr).astype(f32)
        keep = m2
        s0 = jnp.sum(m2[:N], axis=0, keepdims=True)
        s1 = jnp.sum(m2[N:], axis=0, keepdims=True)
        done = jnp.logical_and(jnp.all(s0 < 2.0), jnp.all(s1 < 2.0))

        p1m = pt - (1.0 - m2) * (-NEG)
        v0 = jnp.max(p1m[:N], axis=0, keepdims=True)
        v1 = jnp.max(p1m[N:], axis=0, keepdims=True)
        m3 = jnp.concatenate([(p1m[:N] >= v0).astype(f32),
                              (p1m[N:] >= v1).astype(f32)], axis=0)
        m4 = m3 * m2
        s4_0 = jnp.sum(m4[:N], axis=0, keepdims=True)
        s4_1 = jnp.sum(m4[N:], axis=0, keepdims=True)
        m5 = jnp.concatenate(
            [jnp.broadcast_to((s4_0 > 0.5).astype(f32), (N, L)),
             jnp.broadcast_to((s4_1 > 0.5).astype(f32), (N, L))], axis=0)
        pt2 = pt + m5 * NEG
        m6 = jnp.concatenate(
            [jnp.broadcast_to((s4_0 > 1.5).astype(f32), (N, L)),
             jnp.broadcast_to((s4_1 > 1.5).astype(f32), (N, L))], axis=0)
        m4 = m4 * (1.0 - m6)
        pt = jnp.where(done, pt, m4 * pt + (1.0 - m4) * pt2)
        return j + 1, pt, keep, done

    init = (jnp.zeros((), jnp.int32), pt_stack,
            jnp.zeros((2 * N, L), f32), jnp.zeros((), jnp.bool_))
    _, _, keep, _ = lax.while_loop(cond, body, init)
    return keep


def _body(x_ref, lab_ref, lt_ref, wq_ref, bq_ref, wk_ref, bk_ref, wv_ref,
          bv_ref, wo_ref, bo_ref, out_ref):
    f32 = jnp.float32
    Wq, Wk, Wv, Wo = wq_ref[...], wk_ref[...], wv_ref[...], wo_ref[...]
    bq, bk, bv, bo = bq_ref[...], bk_ref[...], bv_ref[...], bo_ref[...]

    row_i = lax.broadcasted_iota(jnp.int32, (L, L), 0)
    col_i = lax.broadcasted_iota(jnp.int32, (L, L), 1)
    eye = (row_i == col_i).astype(f32)

    sds, rds, graphs, labels, lts = [], [], [], [], []
    for b in range(B):
        x = x_ref[b]
        q = jnp.dot(x, Wq, preferred_element_type=f32) + bq
        k = jnp.dot(x, Wk, preferred_element_type=f32) + bk
        v = jnp.dot(x, Wv, preferred_element_type=f32) + bv
        ctxs = []
        for h in range(NH):
            sl = slice(h * HS, (h + 1) * HS)
            s = lax.dot_general(q[:, sl], k[:, sl],
                                (((1,), (1,)), ((), ())),
                                preferred_element_type=f32)
            p = _softmax_rows(s)
            ctxs.append(jnp.dot(p, v[:, sl], preferred_element_type=f32))
        ctx = jnp.concatenate(ctxs, axis=1)
        g = jnp.dot(ctx, Wo, preferred_element_type=f32) + bo

        sd = jnp.concatenate([g[:L], x[:L]], axis=1)
        rd = jnp.concatenate([g[L:], x[L:]], axis=1)

        gcols = []
        for j in range(0, L, GCHUNK):
            d = sd[:, None, :] - sd[None, j:j + GCHUNK, :]
            gcols.append(-jnp.sqrt(jnp.sum(d * d, axis=-1) + 1e-06))
        graph = jnp.concatenate(gcols, axis=1) + eye * NEG

        sds.append(sd)
        rds.append(rd)
        graphs.append(graph)
        labels.append(lab_ref[b])
        lts.append(lt_ref[b])

    psums = [jnp.zeros((L, N), f32) for _ in range(B)]
    c1 = (K + 1.0) / (K + 2.0)
    c2 = 1.0 / (K + 2.0)

    for it in range(I_N):
        preds = []
        for b in range(B):
            lab, lt = labels[b], lts[b]
            mask = lax.dot_general(lab, lab, (((1,), (1,)), ((), ())),
                                   preferred_element_type=f32)
            dis = graphs[b] * mask + (1.0 - mask) * NEG
            sample_w = jnp.max(dis, axis=0, keepdims=True)
            ltmp = lt * sample_w + (1.0 - lt) * NEG
            probs = _softmax_rows(ltmp)
            center = c1 * jnp.dot(probs, sds[b],
                                  preferred_element_type=f32) + c2 * rds[b]
            d = sds[b][:, None, :] - center[None, :, :]
            pred = -jnp.sum(d * d, axis=-1)
            preds.append(pred)
            psums[b] = psums[b] + pred

        if it == I_N - 1:
            break

        pt_stack = jnp.concatenate(
            [jnp.transpose(preds[0]), jnp.transpose(preds[1])], axis=0)
        keep = _to_one_hot_stacked(pt_stack)
        lts = [keep[:N], keep[N:]]
        labels = [jnp.transpose(keep[:N]), jnp.transpose(keep[N:])]

    for b in range(B):
        out_ref[b] = psums[b] * (1.0 / I_N)


@jax.jit
def kernel(samples, relation, label, Wq, bq, Wk, bk, Wv, bv, Wo, bo):
    x = jnp.concatenate([samples, relation], axis=1)
    lt = jnp.swapaxes(label, 1, 2)
    return pl.pallas_call(
        _body,
        out_shape=jax.ShapeDtypeStruct((B, L, N), jnp.float32),
    )(x, label, lt, Wq, bq.reshape(1, HIDDEN), Wk, bk.reshape(1, HIDDEN),
      Wv, bv.reshape(1, HIDDEN), Wo, bo.reshape(1, HIDDEN))


# speedup vs baseline: 2.9343x; 1.1285x over previous
import jax
import jax.numpy as jnp
from jax import lax
from jax.experimental import pallas as pl

B, L, N = 2, 256, 16
HIDDEN = 128
NH = 4
D_IN = HIDDEN * 2
I_N = 4
HS = HIDDEN // NH
T_TOK = L + N
K = L // N - 1
NP = K + 1
NEG = -1000000.0
BIG_NEG = -3.0e38
GCHUNK = 16


def _softmax_rows(x):
    m = jnp.max(x, axis=1, keepdims=True)
    e = jnp.exp(x - m)
    return e / jnp.sum(e, axis=1, keepdims=True)


def _kth_threshold(pt):
    def step(_, carry):
        m, cum, t = carry
        cur = jnp.max(m, axis=1, keepdims=True)
        t = jnp.where(cum < (NP - 0.5), cur, t)
        eq = m == cur
        cum = cum + jnp.sum(eq.astype(jnp.float32), axis=1, keepdims=True)
        m = jnp.where(eq, BIG_NEG, m)
        return m, cum, t

    rows = pt.shape[0]
    init = (pt, jnp.zeros((rows, 1), jnp.float32),
            jnp.full((rows, 1), BIG_NEG, jnp.float32))
    _, _, t = lax.fori_loop(0, NP, step, init)
    return t


def _to_one_hot_stacked(pt_stack):
    f32 = jnp.float32

    def cond(carry):
        j, _, _, done = carry
        return jnp.logical_and(j < 11, jnp.logical_not(done))

    def body(carry):
        j, pt, _, done = carry
        thr = _kth_threshold(pt)
        m2 = (pt >= thr).astype(f32)
        keep = m2
        s0 = jnp.sum(m2[:N], axis=0, keepdims=True)
        s1 = jnp.sum(m2[N:], axis=0, keepdims=True)
        done = jnp.logical_and(jnp.all(s0 < 2.0), jnp.all(s1 < 2.0))

        p1m = pt - (1.0 - m2) * (-NEG)
        v0 = jnp.max(p1m[:N], axis=0, keepdims=True)
        v1 = jnp.max(p1m[N:], axis=0, keepdims=True)
        m3 = jnp.concatenate([(p1m[:N] >= v0).astype(f32),
                              (p1m[N:] >= v1).astype(f32)], axis=0)
        m4 = m3 * m2
        s4_0 = jnp.sum(m4[:N], axis=0, keepdims=True)
        s4_1 = jnp.sum(m4[N:], axis=0, keepdims=True)
        m5 = jnp.concatenate(
            [jnp.broadcast_to((s4_0 > 0.5).astype(f32), (N, L)),
             jnp.broadcast_to((s4_1 > 0.5).astype(f32), (N, L))], axis=0)
        pt2 = pt + m5 * NEG
        m6 = jnp.concatenate(
            [jnp.broadcast_to((s4_0 > 1.5).astype(f32), (N, L)),
             jnp.broadcast_to((s4_1 > 1.5).astype(f32), (N, L))], axis=0)
        m4 = m4 * (1.0 - m6)
        pt = jnp.where(done, pt, m4 * pt + (1.0 - m4) * pt2)
        return j + 1, pt, keep, done

    init = (jnp.zeros((), jnp.int32), pt_stack,
            jnp.zeros((2 * N, L), f32), jnp.zeros((), jnp.bool_))
    _, _, keep, _ = lax.while_loop(cond, body, init)
    return keep


def _body(x_ref, lab_ref, lt_ref, wq_ref, bq_ref, wk_ref, bk_ref, wv_ref,
          bv_ref, wo_ref, bo_ref, out_ref):
    f32 = jnp.float32
    Wq, Wk, Wv, Wo = wq_ref[...], wk_ref[...], wv_ref[...], wo_ref[...]
    bq, bk, bv, bo = bq_ref[...], bk_ref[...], bv_ref[...], bo_ref[...]

    row_i = lax.broadcasted_iota(jnp.int32, (L, L), 0)
    col_i = lax.broadcasted_iota(jnp.int32, (L, L), 1)
    eye = (row_i == col_i).astype(f32)

    sds, rds, graphs, labels, lts = [], [], [], [], []
    for b in range(B):
        x = x_ref[b]
        q = jnp.dot(x, Wq, preferred_element_type=f32) + bq
        k = jnp.dot(x, Wk, preferred_element_type=f32) + bk
        v = jnp.dot(x, Wv, preferred_element_type=f32) + bv
        ctxs = []
        for h in range(NH):
            sl = slice(h * HS, (h + 1) * HS)
            s = lax.dot_general(q[:, sl], k[:, sl],
                                (((1,), (1,)), ((), ())),
                                preferred_element_type=f32)
            p = _softmax_rows(s)
            ctxs.append(jnp.dot(p, v[:, sl], preferred_element_type=f32))
        ctx = jnp.concatenate(ctxs, axis=1)
        g = jnp.dot(ctx, Wo, preferred_element_type=f32) + bo

        sd = jnp.concatenate([g[:L], x[:L]], axis=1)
        rd = jnp.concatenate([g[L:], x[L:]], axis=1)

        gcols = []
        for j in range(0, L, GCHUNK):
            d = sd[j:, None, :] - sd[None, j:j + GCHUNK, :]
            low = -jnp.sqrt(jnp.sum(d * d, axis=-1) + 1e-06)
            if j:
                low = jnp.concatenate(
                    [jnp.zeros((j, GCHUNK), f32), low], axis=0)
            gcols.append(low)
        graph_low = jnp.concatenate(gcols, axis=1)
        lower = row_i >= col_i
        graph = (jnp.where(lower, graph_low, jnp.transpose(graph_low))
                 + eye * NEG)

        sds.append(sd)
        rds.append(rd)
        graphs.append(graph)
        labels.append(lab_ref[b])
        lts.append(lt_ref[b])

    psums = [jnp.zeros((L, N), f32) for _ in range(B)]
    c1 = (K + 1.0) / (K + 2.0)
    c2 = 1.0 / (K + 2.0)

    for it in range(I_N):
        preds = []
        for b in range(B):
            lab, lt = labels[b], lts[b]
            mask = lax.dot_general(lab, lab, (((1,), (1,)), ((), ())),
                                   preferred_element_type=f32)
            dis = graphs[b] * mask + (1.0 - mask) * NEG
            sample_w = jnp.max(dis, axis=0, keepdims=True)
            ltmp = lt * sample_w + (1.0 - lt) * NEG
            probs = _softmax_rows(ltmp)
            center = c1 * jnp.dot(probs, sds[b],
                                  preferred_element_type=f32) + c2 * rds[b]
            d = sds[b][:, None, :] - center[None, :, :]
            pred = -jnp.sum(d * d, axis=-1)
            preds.append(pred)
            psums[b] = psums[b] + pred

        if it == I_N - 1:
            break

        pt_stack = jnp.concatenate(
            [jnp.transpose(preds[0]), jnp.transpose(preds[1])], axis=0)
        keep = _to_one_hot_stacked(pt_stack)
        lts = [keep[:N], keep[N:]]
        labels = [jnp.transpose(keep[:N]), jnp.transpose(keep[N:])]

    for b in range(B):
        out_ref[b] = psums[b] * (1.0 / I_N)


@jax.jit
def kernel(samples, relation, label, Wq, bq, Wk, bk, Wv, bv, Wo, bo):
    x = jnp.concatenate([samples, relation], axis=1)
    lt = jnp.swapaxes(label, 1, 2)
    return pl.pallas_call(
        _body,
        out_shape=jax.ShapeDtypeStruct((B, L, N), jnp.float32),
    )(x, label, lt, Wq, bq.reshape(1, HIDDEN), Wk, bk.reshape(1, HIDDEN),
      Wv, bv.reshape(1, HIDDEN), Wo, bo.reshape(1, HIDDEN))


# speedup vs baseline: 4.0466x; 1.3791x over previous
import jax
import jax.numpy as jnp
from jax import lax
from jax.experimental import pallas as pl

B, L, N = 2, 256, 16
HIDDEN = 128
NH = 4
D_IN = HIDDEN * 2
I_N = 4
HS = HIDDEN // NH
T_TOK = L + N
K = L // N - 1
NP = K + 1
NEG = -1000000.0
BIG_NEG = -3.0e38
GCHUNK = 16


def _softmax_rows(x):
    m = jnp.max(x, axis=1, keepdims=True)
    e = jnp.exp(x - m)
    return e / jnp.sum(e, axis=1, keepdims=True)


def _kth_threshold(pt):
    rows = pt.shape[0]
    m = pt
    cum = jnp.zeros((rows, 1), jnp.float32)
    t = jnp.full((rows, 1), BIG_NEG, jnp.float32)
    for _ in range(NP):
        cur = jnp.max(m, axis=1, keepdims=True)
        t = jnp.where(cum < (NP - 0.5), cur, t)
        eq = m == cur
        cum = cum + jnp.sum(eq.astype(jnp.float32), axis=1, keepdims=True)
        m = jnp.where(eq, BIG_NEG, m)
    return t


def _to_one_hot_stacked(pt_stack):
    f32 = jnp.float32

    def cond(carry):
        j, _, _, done = carry
        return jnp.logical_and(j < 11, jnp.logical_not(done))

    def body(carry):
        j, pt, _, done = carry
        thr = _kth_threshold(pt)
        m2 = (pt >= thr).astype(f32)
        keep = m2
        s0 = jnp.sum(m2[:N], axis=0, keepdims=True)
        s1 = jnp.sum(m2[N:], axis=0, keepdims=True)
        done = jnp.logical_and(jnp.all(s0 < 2.0), jnp.all(s1 < 2.0))

        p1m = pt - (1.0 - m2) * (-NEG)
        v0 = jnp.max(p1m[:N], axis=0, keepdims=True)
        v1 = jnp.max(p1m[N:], axis=0, keepdims=True)
        m3 = jnp.concatenate([(p1m[:N] >= v0).astype(f32),
                              (p1m[N:] >= v1).astype(f32)], axis=0)
        m4 = m3 * m2
        s4_0 = jnp.sum(m4[:N], axis=0, keepdims=True)
        s4_1 = jnp.sum(m4[N:], axis=0, keepdims=True)
        m5 = jnp.concatenate(
            [jnp.broadcast_to((s4_0 > 0.5).astype(f32), (N, L)),
             jnp.broadcast_to((s4_1 > 0.5).astype(f32), (N, L))], axis=0)
        pt2 = pt + m5 * NEG
        m6 = jnp.concatenate(
            [jnp.broadcast_to((s4_0 > 1.5).astype(f32), (N, L)),
             jnp.broadcast_to((s4_1 > 1.5).astype(f32), (N, L))], axis=0)
        m4 = m4 * (1.0 - m6)
        pt = jnp.where(done, pt, m4 * pt + (1.0 - m4) * pt2)
        return j + 1, pt, keep, done

    init = (jnp.zeros((), jnp.int32), pt_stack,
            jnp.zeros((2 * N, L), f32), jnp.zeros((), jnp.bool_))
    _, _, keep, _ = lax.while_loop(cond, body, init)
    return keep


def _body(x_ref, lab_ref, lt_ref, wq_ref, bq_ref, wk_ref, bk_ref, wv_ref,
          bv_ref, wo_ref, bo_ref, out_ref):
    f32 = jnp.float32
    Wq, Wk, Wv, Wo = wq_ref[...], wk_ref[...], wv_ref[...], wo_ref[...]
    bq, bk, bv, bo = bq_ref[...], bk_ref[...], bv_ref[...], bo_ref[...]

    row_i = lax.broadcasted_iota(jnp.int32, (L, L), 0)
    col_i = lax.broadcasted_iota(jnp.int32, (L, L), 1)
    eye = (row_i == col_i).astype(f32)

    sds, rds, graphs, labels, lts = [], [], [], [], []
    for b in range(B):
        x = x_ref[b]
        q = jnp.dot(x, Wq, preferred_element_type=f32) + bq
        k = jnp.dot(x, Wk, preferred_element_type=f32) + bk
        v = jnp.dot(x, Wv, preferred_element_type=f32) + bv
        ctxs = []
        for h in range(NH):
            sl = slice(h * HS, (h + 1) * HS)
            s = lax.dot_general(q[:, sl], k[:, sl],
                                (((1,), (1,)), ((), ())),
                                preferred_element_type=f32)
            p = _softmax_rows(s)
            ctxs.append(jnp.dot(p, v[:, sl], preferred_element_type=f32))
        ctx = jnp.concatenate(ctxs, axis=1)
        g = jnp.dot(ctx, Wo, preferred_element_type=f32) + bo

        sd = jnp.concatenate([g[:L], x[:L]], axis=1)
        rd = jnp.concatenate([g[L:], x[L:]], axis=1)

        gcols = []
        for j in range(0, L, GCHUNK):
            d = sd[j:, None, :] - sd[None, j:j + GCHUNK, :]
            low = -jnp.sqrt(jnp.sum(d * d, axis=-1) + 1e-06)
            if j:
                low = jnp.concatenate(
                    [jnp.zeros((j, GCHUNK), f32), low], axis=0)
            gcols.append(low)
        graph_low = jnp.concatenate(gcols, axis=1)
        lower = row_i >= col_i
        graph = (jnp.where(lower, graph_low, jnp.transpose(graph_low))
                 + eye * NEG)

        sds.append(sd)
        rds.append(rd)
        graphs.append(graph)
        labels.append(lab_ref[b])
        lts.append(lt_ref[b])

    psums = [jnp.zeros((L, N), f32) for _ in range(B)]
    c1 = (K + 1.0) / (K + 2.0)
    c2 = 1.0 / (K + 2.0)

    for it in range(I_N):
        preds = []
        for b in range(B):
            lab, lt = labels[b], lts[b]
            mask = lax.dot_general(lab, lab, (((1,), (1,)), ((), ())),
                                   preferred_element_type=f32)
            dis = graphs[b] * mask + (1.0 - mask) * NEG
            sample_w = jnp.max(dis, axis=0, keepdims=True)
            ltmp = lt * sample_w + (1.0 - lt) * NEG
            probs = _softmax_rows(ltmp)
            center = c1 * jnp.dot(probs, sds[b],
                                  preferred_element_type=f32) + c2 * rds[b]
            d = sds[b][:, None, :] - center[None, :, :]
            pred = -jnp.sum(d * d, axis=-1)
            preds.append(pred)
            psums[b] = psums[b] + pred

        if it == I_N - 1:
            break

        pt_stack = jnp.concatenate(
            [jnp.transpose(preds[0]), jnp.transpose(preds[1])], axis=0)
        keep = _to_one_hot_stacked(pt_stack)
        lts = [keep[:N], keep[N:]]
        labels = [jnp.transpose(keep[:N]), jnp.transpose(keep[N:])]

    for b in range(B):
        out_ref[b] = psums[b] * (1.0 / I_N)


@jax.jit
def kernel(samples, relation, label, Wq, bq, Wk, bk, Wv, bv, Wo, bo):
    x = jnp.concatenate([samples, relation], axis=1)
    lt = jnp.swapaxes(label, 1, 2)
    return pl.pallas_call(
        _body,
        out_shape=jax.ShapeDtypeStruct((B, L, N), jnp.float32),
    )(x, label, lt, Wq, bq.reshape(1, HIDDEN), Wk, bk.reshape(1, HIDDEN),
      Wv, bv.reshape(1, HIDDEN), Wo, bo.reshape(1, HIDDEN))
